# baseline (speedup 1.0000x reference)
import functools

import jax
import jax.numpy as jnp
from jax import lax
from jax.experimental import pallas as pl
from jax.experimental.pallas import tpu as pltpu
from jax.experimental.pallas import tpu_sc as plsc

NC, NS, L = 2, 16, 16
NW = NC * NS
BSZ, SEQ, D = 4, 8192, 1024
POS_PER_W = SEQ // NW
CHUNK = 32
NCHUNK = POS_PER_W // CHUNK


def _body(ids_hbm, w_hbm, pe_hbm, out_hbm, pos_hbm,
          idx_v, pe_v, tok_v, pos_v, sem):
  wid = lax.axis_index("s") * NC + lax.axis_index("c")
  p0 = wid * POS_PER_W

  @pl.loop(0, POS_PER_W // L)
  def _(i):
    pos_v[pl.ds(i * L, L)] = p0 + i * L + lax.iota(jnp.int32, L)

  for b in range(BSZ):
    pltpu.sync_copy(pos_v, pos_hbm.at[pl.ds(b * SEQ + p0, POS_PER_W)])

  for k in range(NCHUNK):
    base = k * CHUNK
    pltpu.sync_copy(pe_hbm.at[pl.ds(p0 + base, CHUNK), :], pe_v)
    for b in range(BSZ):
      pltpu.sync_copy(ids_hbm.at[pl.ds(b * SEQ + p0 + base, CHUNK)], idx_v)
      pltpu.async_copy(w_hbm.at[idx_v], tok_v, sem).wait()

      @pl.loop(0, CHUNK)
      def _(r):
        @pl.loop(0, D // L)
        def _(j):
          sl = pl.ds(j * L, L)
          tok_v[r, sl] = tok_v[r, sl] + pe_v[r, sl]

      pltpu.sync_copy(tok_v, out_hbm.at[pl.ds(b * SEQ + p0 + base, CHUNK), :])


@functools.partial(jax.jit, static_argnames=())
def _run(ids_flat, W, pe):
  mesh = plsc.VectorSubcoreMesh(core_axis_name="c", subcore_axis_name="s",
                                num_cores=NC, num_subcores=NS)
  out_type = (jax.ShapeDtypeStruct((BSZ * SEQ, D), jnp.float32),
              jax.ShapeDtypeStruct((BSZ * SEQ,), jnp.int32))
  scratch = [
      pltpu.VMEM((CHUNK,), jnp.int32),
      pltpu.VMEM((CHUNK, D), jnp.float32),
      pltpu.VMEM((CHUNK, D), jnp.float32),
      pltpu.VMEM((POS_PER_W,), jnp.int32),
      pltpu.SemaphoreType.DMA,
  ]
  return pl.kernel(_body, out_type=out_type, mesh=mesh,
                   scratch_types=scratch)(ids_flat, W, pe)


def kernel(input_ids, W, pe):
  ids_flat = input_ids.reshape(-1).astype(jnp.int32)
  x, pos = _run(ids_flat, W, pe)
  return (x.reshape(BSZ, SEQ, D),
          pos.reshape(BSZ, SEQ).astype(input_ids.dtype))


# speedup vs baseline: 1.2562x; 1.2562x over previous
import functools

import jax
import jax.numpy as jnp
from jax import lax
from jax.experimental import pallas as pl
from jax.experimental.pallas import tpu as pltpu
from jax.experimental.pallas import tpu_sc as plsc

NC, NS, L = 2, 16, 16
NW = NC * NS
BSZ, SEQ, D = 4, 8192, 1024
POS_PER_W = SEQ // NW
CHUNK = 32
NCHUNK = POS_PER_W // CHUNK
NITEM = NCHUNK * BSZ


def _body(ids_hbm, w_hbm, pe_hbm, out_hbm, pos_hbm,
          idx_v, pe_v, tok0, tok1, pos_v, gsem0, gsem1, ssem0, ssem1):
  toks = (tok0, tok1)
  gsems = (gsem0, gsem1)
  ssems = (ssem0, ssem1)
  wid = lax.axis_index("s") * NC + lax.axis_index("c")
  p0 = wid * POS_PER_W

  @pl.loop(0, POS_PER_W // L)
  def _(i):
    pos_v[pl.ds(i * L, L)] = p0 + i * L + lax.iota(jnp.int32, L)

  for b in range(BSZ):
    pltpu.sync_copy(pos_v, pos_hbm.at[pl.ds(b * SEQ + p0, POS_PER_W)])

  for b in range(BSZ):
    pltpu.sync_copy(ids_hbm.at[pl.ds(b * SEQ + p0, POS_PER_W)],
                    idx_v.at[pl.ds(b * POS_PER_W, POS_PER_W)])

  def item_idx(t):
    k, b = divmod(t, BSZ)
    return b * POS_PER_W + k * CHUNK

  def out_off(t):
    k, b = divmod(t, BSZ)
    return b * SEQ + p0 + k * CHUNK

  def start_gather(t):
    s = t % 2
    pltpu.async_copy(w_hbm.at[idx_v.at[pl.ds(item_idx(t), CHUNK)]],
                     toks[s], gsems[s])

  pltpu.sync_copy(pe_hbm.at[pl.ds(p0, CHUNK), :], pe_v)
  start_gather(0)

  for t in range(NITEM):
    s = t % 2
    if t + 1 < NITEM:
      if t + 1 >= 2:
        pltpu.make_async_copy(toks[(t + 1) % 2],
                              out_hbm.at[pl.ds(out_off(t - 1), CHUNK), :],
                              ssems[(t + 1) % 2]).wait()
      start_gather(t + 1)
    pltpu.make_async_copy(w_hbm.at[idx_v.at[pl.ds(item_idx(t), CHUNK)]],
                          toks[s], gsems[s]).wait()

    tok = toks[s]

    @pl.loop(0, CHUNK)
    def _(r):
      @pl.loop(0, D // L)
      def _(j):
        sl = pl.ds(j * L, L)
        tok[r, sl] = tok[r, sl] + pe_v[r, sl]

    pltpu.async_copy(tok, out_hbm.at[pl.ds(out_off(t), CHUNK), :], ssems[s])

    if t + 1 < NITEM and (t + 1) % BSZ == 0:
      pltpu.sync_copy(pe_hbm.at[pl.ds(p0 + ((t + 1) // BSZ) * CHUNK, CHUNK), :],
                      pe_v)

  pltpu.make_async_copy(toks[(NITEM - 2) % 2],
                        out_hbm.at[pl.ds(out_off(NITEM - 2), CHUNK), :],
                        ssems[(NITEM - 2) % 2]).wait()
  pltpu.make_async_copy(toks[(NITEM - 1) % 2],
                        out_hbm.at[pl.ds(out_off(NITEM - 1), CHUNK), :],
                        ssems[(NITEM - 1) % 2]).wait()


@jax.jit
def _run(ids_flat, W, pe):
  mesh = plsc.VectorSubcoreMesh(core_axis_name="c", subcore_axis_name="s",
                                num_cores=NC, num_subcores=NS)
  out_type = (jax.ShapeDtypeStruct((BSZ * SEQ, D), jnp.float32),
              jax.ShapeDtypeStruct((BSZ * SEQ,), jnp.int32))
  scratch = [
      pltpu.VMEM((BSZ * POS_PER_W,), jnp.int32),
      pltpu.VMEM((CHUNK, D), jnp.float32),
      pltpu.VMEM((CHUNK, D), jnp.float32),
      pltpu.VMEM((CHUNK, D), jnp.float32),
      pltpu.VMEM((POS_PER_W,), jnp.int32),
      pltpu.SemaphoreType.DMA,
      pltpu.SemaphoreType.DMA,
      pltpu.SemaphoreType.DMA,
      pltpu.SemaphoreType.DMA,
  ]
  return pl.kernel(_body, out_type=out_type, mesh=mesh,
                   scratch_types=scratch)(ids_flat, W, pe)


def kernel(input_ids, W, pe):
  ids_flat = input_ids.reshape(-1).astype(jnp.int32)
  x, pos = _run(ids_flat, W, pe)
  return (x.reshape(BSZ, SEQ, D),
          pos.reshape(BSZ, SEQ).astype(input_ids.dtype))
